# baseline (speedup 1.0000x reference)
import functools

import jax
import jax.numpy as jnp
from jax import lax
from jax.experimental import pallas as pl
from jax.experimental.pallas import tpu as pltpu
from jax.experimental.pallas import tpu_sc as plsc

_NUM_CLASSES = 20
_NUM_ANC = 5
_GRID = 13
_BATCH = 16
_NBOX = 8
_NB = _BATCH * _NBOX
_S = 1 + _NUM_CLASSES
_NCELL = _GRID * _GRID
_NROW = _NCELL * _BATCH
_NCH = (_S + 4) * _NUM_ANC
_EPS16 = 0.0009765625
_ANCW = (0.074, 0.147, 0.282, 0.471, 0.784)
_ANCH = (0.060, 0.151, 0.231, 0.425, 0.740)

_DNT = (((1,), (1,)), ((), ()))


def _sc_body(ptp_hbm, gb_hbm, gat_hbm, gb_v, idx_v, rows_v, sem):
    i32 = jnp.int32
    wid = lax.axis_index("c") * 16 + lax.axis_index("s")

    @pl.when(wid == 0)
    def _():
        pltpu.sync_copy(gb_hbm, gb_v)
        lane = lax.iota(i32, 16)
        for i in range(_NBOX):
            lft = gb_v[i, 0, :]
            top = gb_v[i, 1, :]
            rgt = gb_v[i, 2, :]
            btm = gb_v[i, 3, :]
            xsx = (lft + rgt) * (0.5 * _GRID)
            xsy = (top + btm) * (0.5 * _GRID)
            col = xsx.astype(i32)
            row = xsy.astype(i32)
            cell = row * _GRID + col
            idx_v[pl.ds(i * 16, 16)] = cell * _BATCH + lane
        pltpu.async_copy(ptp_hbm.at[idx_v], rows_v, sem).wait()
        pltpu.sync_copy(rows_v, gat_hbm)


_sc_gather = functools.partial(
    pl.kernel,
    out_type=jax.ShapeDtypeStruct((_NB, 128), jnp.float32),
    mesh=plsc.VectorSubcoreMesh(core_axis_name="c", subcore_axis_name="s"),
    scratch_types=[
        pltpu.VMEM((_NBOX, 4, _BATCH), jnp.float32),
        pltpu.VMEM((_NB,), jnp.int32),
        pltpu.VMEM((_NB, 128), jnp.float32),
        pltpu.SemaphoreType.DMA,
    ],
)(_sc_body)


def _tc_body(pt_ref, gb_ref, gl_ref, gat_ref, out_ref):
    f32 = jnp.float32
    i32 = jnp.int32
    lft = gb_ref[:, 0, :]
    top = gb_ref[:, 1, :]
    rgt = gb_ref[:, 2, :]
    btm = gb_ref[:, 3, :]
    glint = gl_ref[...]

    w_g = rgt - lft
    h_g = btm - top
    area_g = w_g * h_g
    best = jnp.full(w_g.shape, -1.0, f32)
    ids = jnp.zeros(w_g.shape, i32)
    for a in range(_NUM_ANC):
        inter = jnp.minimum(w_g, _ANCW[a]) * jnp.minimum(h_g, _ANCH[a])
        iou = inter / (area_g + _ANCW[a] * _ANCH[a] - inter)
        upd = iou > best
        ids = jnp.where(upd, a, ids)
        best = jnp.maximum(iou, best)

    xsx = (lft + rgt) * (0.5 * _GRID)
    xsy = (top + btm) * (0.5 * _GRID)
    flx = jnp.floor(xsx)
    fly = jnp.floor(xsy)
    txx = xsx - flx
    txyy = xsy - fly
    cell = fly.astype(i32) * _GRID + flx.astype(i32)
    aw = jnp.full(w_g.shape, _ANCW[-1], f32)
    ah = jnp.full(w_g.shape, _ANCH[-1], f32)
    for a in range(_NUM_ANC - 2, -1, -1):
        aw = jnp.where(ids == a, _ANCW[a], aw)
        ah = jnp.where(ids == a, _ANCH[a], ah)
    twx = jnp.log(w_g / aw)
    twy = jnp.log(h_g / ah)
    wgt = 2.0 - area_g

    ii = jax.lax.broadcasted_iota(i32, (_NBOX, _NBOX, _BATCH), 0)
    jj = jax.lax.broadcasted_iota(i32, (_NBOX, _NBOX, _BATCH), 1)
    clob = (cell[None, :, :] == cell[:, None, :]) & (jj > ii)
    last = jnp.where(jnp.any(clob, axis=1), 0.0, 1.0)

    gat = jnp.reshape(gat_ref[...], (_NBOX, _BATCH, 128))[:, :, :_NCH]
    pt = pt_ref[...]

    kio5 = jax.lax.broadcasted_iota(i32, (_NUM_ANC, _NCH), 1)
    ai5 = jax.lax.broadcasted_iota(i32, (_NUM_ANC, _NCH), 0)
    eye5 = (kio5 == ai5).astype(f32)
    confT = jax.lax.dot_general(eye5, pt, _DNT, preferred_element_type=f32)
    pcA = jnp.clip(jax.nn.sigmoid(confT), 1e-7, 1.0 - 1e-7)
    fnegA = -0.5 * pcA * pcA * jnp.log(1.0 - pcA)
    ksum = jnp.sum(fnegA, axis=0, keepdims=True)
    rio = jax.lax.broadcasted_iota(i32, (_BATCH, _NROW), 1)
    bio = jax.lax.broadcasted_iota(i32, (_BATCH, _NROW), 0)
    selt = ((rio & (_BATCH - 1)) == bio).astype(f32)
    neg_all = jax.lax.dot_general(ksum, selt, _DNT, preferred_element_type=f32)

    kiota = jax.lax.broadcasted_iota(i32, (_NBOX, _BATCH, _NCH), 2)
    cmap = jnp.floor(kiota.astype(f32) * 0.2).astype(i32)
    amap = kiota - cmap * _NUM_ANC
    am = amap == ids[..., None]
    pcl = jnp.clip(jax.nn.sigmoid(gat), 1e-7, 1.0 - 1e-7)
    logp = jnp.log(pcl)
    log1mp = jnp.log(1.0 - pcl)

    is_conf = cmap == 0
    fneg = -0.5 * pcl * pcl * log1mp
    sub = jnp.sum(jnp.where(is_conf, fneg, 0.0), axis=2)
    fpos = -0.5 * (1.0 - pcl) * (1.0 - pcl) * logp
    cpos = jnp.sum(jnp.where(is_conf & am, fpos, 0.0), axis=2)

    tcls = (cmap == glint[..., None]).astype(f32)
    bce_cls = -(tcls * logp + (1.0 - tcls) * log1mp)
    mcls = (cmap >= 1) & (cmap <= _NUM_CLASSES) & am
    clsv = jnp.sum(jnp.where(mcls, bce_cls, 0.0), axis=2)

    tgxy = jnp.where(cmap == _S, txx[..., None], txyy[..., None])
    bce_xy = -(tgxy * logp + (1.0 - tgxy) * log1mp)
    mxy = ((cmap == _S) | (cmap == _S + 1)) & am
    xyv = jnp.sum(jnp.where(mxy, bce_xy, 0.0), axis=2)

    tgwh = jnp.where(cmap == _S + 2, twx[..., None], twy[..., None])
    dwh = gat - tgwh
    mwh = ((cmap == _S + 2) | (cmap == _S + 3)) & am
    whv = jnp.sum(jnp.where(mwh, dwh * dwh, 0.0), axis=2)

    npos_b = jnp.sum(last, axis=0, keepdims=True)
    sub_i = jnp.sum(last * sub, axis=0, keepdims=True)
    cpos_i = jnp.sum(last * cpos, axis=0, keepdims=True)
    npos_t = jnp.maximum(jnp.sum(npos_b), 1.0)
    nneg_b = jnp.maximum(float(_NCELL * _NUM_ANC) - 5.0 * npos_b, _EPS16)
    npos_bc = jnp.maximum(npos_b, _EPS16)
    l_conf_neg = jnp.sum((neg_all - sub_i) / nneg_b) * (3.0 / _BATCH)
    l_conf_pos = jnp.sum(cpos_i / npos_bc) / _BATCH
    l_cls = jnp.sum(last * clsv) / npos_t
    l_txty = jnp.sum(last * wgt * xyv) / npos_t
    l_twth = jnp.sum(last * wgt * whv) / npos_t

    total = l_conf_pos + l_conf_neg + l_cls + l_txty + l_twth
    out_ref[...] = jnp.broadcast_to(total, (1, 1))


def kernel(pyolos, gboxes_ltrb, glabels):
    pt = pyolos.transpose(2, 3, 0, 1).reshape(_NROW, _NCH)
    ptp = jnp.pad(pt, ((0, 0), (0, 128 - _NCH)))
    gb = gboxes_ltrb.astype(jnp.float32).transpose(1, 2, 0)
    gl = glabels.astype(jnp.int32).transpose(1, 0)
    gat = _sc_gather(ptp, gb)
    out = pl.pallas_call(
        _tc_body,
        out_shape=jax.ShapeDtypeStruct((1, 1), jnp.float32),
    )(pt, gb, gl, gat)
    return out[0, 0]


# speedup vs baseline: 6.6650x; 6.6650x over previous
import jax
import jax.numpy as jnp
from jax.experimental import pallas as pl
from jax.experimental.pallas import tpu as pltpu

_NUM_CLASSES = 20
_NUM_ANC = 5
_GRID = 13
_BATCH = 16
_NBOX = 8
_NB = _BATCH * _NBOX
_S = 1 + _NUM_CLASSES
_NCELL = _GRID * _GRID
_NROW = _NCELL * _BATCH
_NCH = (_S + 4) * _NUM_ANC
_CHD = _NROW // 2
_EPS16 = 0.0009765625
_ANCW = (0.074, 0.147, 0.282, 0.471, 0.784)
_ANCH = (0.060, 0.151, 0.231, 0.425, 0.740)

_DNT = (((1,), (1,)), ((), ()))


def _loss_body(pt_hbm, gb_ref, gl_ref, out_ref, buf, sem0, sem1):
    f32 = jnp.float32
    i32 = jnp.int32
    cp0 = pltpu.make_async_copy(pt_hbm.at[pl.ds(0, _CHD), :], buf.at[0], sem0)
    cp1 = pltpu.make_async_copy(pt_hbm.at[pl.ds(_CHD, _CHD), :], buf.at[1],
                                sem1)
    cp0.start()
    cp1.start()

    lft = gb_ref[:, 0, :]
    top = gb_ref[:, 1, :]
    rgt = gb_ref[:, 2, :]
    btm = gb_ref[:, 3, :]
    glint = gl_ref[...]

    w_g = rgt - lft
    h_g = btm - top
    area_g = w_g * h_g
    best = jnp.full(w_g.shape, -1.0, f32)
    ids = jnp.zeros(w_g.shape, i32)
    for a in range(_NUM_ANC):
        inter = jnp.minimum(w_g, _ANCW[a]) * jnp.minimum(h_g, _ANCH[a])
        iou = inter / (area_g + _ANCW[a] * _ANCH[a] - inter)
        upd = iou > best
        ids = jnp.where(upd, a, ids)
        best = jnp.maximum(iou, best)

    xsx = (lft + rgt) * (0.5 * _GRID)
    xsy = (top + btm) * (0.5 * _GRID)
    flx = jnp.floor(xsx)
    fly = jnp.floor(xsy)
    txx = xsx - flx
    txyy = xsy - fly
    cell = fly.astype(i32) * _GRID + flx.astype(i32)
    aw = jnp.full(w_g.shape, _ANCW[-1], f32)
    ah = jnp.full(w_g.shape, _ANCH[-1], f32)
    for a in range(_NUM_ANC - 2, -1, -1):
        aw = jnp.where(ids == a, _ANCW[a], aw)
        ah = jnp.where(ids == a, _ANCH[a], ah)
    twx = jnp.log(w_g / aw)
    twy = jnp.log(h_g / ah)
    wgt = 2.0 - area_g

    ii = jax.lax.broadcasted_iota(i32, (_NBOX, _NBOX, _BATCH), 0)
    jj = jax.lax.broadcasted_iota(i32, (_NBOX, _NBOX, _BATCH), 1)
    clob = (cell[None, :, :] == cell[:, None, :]) & (jj > ii)
    last = jnp.where(jnp.any(clob, axis=1), 0.0, 1.0)

    bimg = jax.lax.broadcasted_iota(i32, (_NBOX, _BATCH), 1)
    hot = cell * _BATCH + bimg
    riota = jax.lax.broadcasted_iota(i32, (_NBOX, _BATCH, _CHD), 2)
    oh0 = jnp.reshape((riota == hot[..., None]).astype(f32), (_NB, _CHD))
    oh1 = jnp.reshape((riota + _CHD == hot[..., None]).astype(f32),
                      (_NB, _CHD))

    kio5 = jax.lax.broadcasted_iota(i32, (_NUM_ANC, _NCH), 1)
    ai5 = jax.lax.broadcasted_iota(i32, (_NUM_ANC, _NCH), 0)
    eye5 = (kio5 == ai5).astype(f32)
    rio = jax.lax.broadcasted_iota(i32, (_BATCH, _CHD), 1)
    bio = jax.lax.broadcasted_iota(i32, (_BATCH, _CHD), 0)
    selt0 = ((rio & (_BATCH - 1)) == bio).astype(f32)
    selt1 = (((rio + _CHD) & (_BATCH - 1)) == bio).astype(f32)

    def _chunk(ptc, oh, selt):
        gat_c = jax.lax.dot_general(oh, ptc, (((1,), (0,)), ((), ())),
                                    preferred_element_type=f32)
        confT = jax.lax.dot_general(eye5, ptc, _DNT,
                                    preferred_element_type=f32)
        pcA = jnp.clip(jax.nn.sigmoid(confT), 1e-7, 1.0 - 1e-7)
        fnegA = -0.5 * pcA * pcA * jnp.log(1.0 - pcA)
        ksum = jnp.sum(fnegA, axis=0, keepdims=True)
        neg_c = jax.lax.dot_general(ksum, selt, _DNT,
                                    preferred_element_type=f32)
        return gat_c, neg_c

    cp0.wait()
    gat_a, neg_a = _chunk(buf[0], oh0, selt0)
    cp1.wait()
    gat_b, neg_b = _chunk(buf[1], oh1, selt1)
    gat = jnp.reshape(gat_a + gat_b, (_NBOX, _BATCH, _NCH))
    neg_all = neg_a + neg_b

    kiota = jax.lax.broadcasted_iota(i32, (_NBOX, _BATCH, _NCH), 2)
    cmap = jnp.floor(kiota.astype(f32) * 0.2).astype(i32)
    amap = kiota - cmap * _NUM_ANC
    am = amap == ids[..., None]
    pcl = jnp.clip(jax.nn.sigmoid(gat), 1e-7, 1.0 - 1e-7)
    logp = jnp.log(pcl)
    log1mp = jnp.log(1.0 - pcl)

    is_conf = cmap == 0
    fneg = -0.5 * pcl * pcl * log1mp
    sub = jnp.sum(jnp.where(is_conf, fneg, 0.0), axis=2)
    fpos = -0.5 * (1.0 - pcl) * (1.0 - pcl) * logp
    cpos = jnp.sum(jnp.where(is_conf & am, fpos, 0.0), axis=2)

    tcls = (cmap == glint[..., None]).astype(f32)
    bce_cls = -(tcls * logp + (1.0 - tcls) * log1mp)
    mcls = (cmap >= 1) & (cmap <= _NUM_CLASSES) & am
    clsv = jnp.sum(jnp.where(mcls, bce_cls, 0.0), axis=2)

    tgxy = jnp.where(cmap == _S, txx[..., None], txyy[..., None])
    bce_xy = -(tgxy * logp + (1.0 - tgxy) * log1mp)
    mxy = ((cmap == _S) | (cmap == _S + 1)) & am
    xyv = jnp.sum(jnp.where(mxy, bce_xy, 0.0), axis=2)

    tgwh = jnp.where(cmap == _S + 2, twx[..., None], twy[..., None])
    dwh = gat - tgwh
    mwh = ((cmap == _S + 2) | (cmap == _S + 3)) & am
    whv = jnp.sum(jnp.where(mwh, dwh * dwh, 0.0), axis=2)

    npos_b = jnp.sum(last, axis=0, keepdims=True)
    sub_i = jnp.sum(last * sub, axis=0, keepdims=True)
    cpos_i = jnp.sum(last * cpos, axis=0, keepdims=True)
    npos_t = jnp.maximum(jnp.sum(npos_b), 1.0)
    nneg_b = jnp.maximum(float(_NCELL * _NUM_ANC) - 5.0 * npos_b, _EPS16)
    npos_bc = jnp.maximum(npos_b, _EPS16)
    l_conf_neg = jnp.sum((neg_all - sub_i) / nneg_b) * (3.0 / _BATCH)
    l_conf_pos = jnp.sum(cpos_i / npos_bc) / _BATCH
    l_cls = jnp.sum(last * clsv) / npos_t
    l_txty = jnp.sum(last * wgt * xyv) / npos_t
    l_twth = jnp.sum(last * wgt * whv) / npos_t

    total = l_conf_pos + l_conf_neg + l_cls + l_txty + l_twth
    out_ref[...] = jnp.broadcast_to(total, (1, 1))


def kernel(pyolos, gboxes_ltrb, glabels):
    pt = pyolos.transpose(2, 3, 0, 1).reshape(_NROW, _NCH)
    gb = gboxes_ltrb.astype(jnp.float32).transpose(1, 2, 0)
    gl = glabels.astype(jnp.int32).transpose(1, 0)
    out = pl.pallas_call(
        _loss_body,
        in_specs=[
            pl.BlockSpec(memory_space=pl.ANY),
            pl.BlockSpec((_NBOX, 4, _BATCH), lambda: (0, 0, 0)),
            pl.BlockSpec((_NBOX, _BATCH), lambda: (0, 0)),
        ],
        out_specs=pl.BlockSpec((1, 1), lambda: (0, 0)),
        out_shape=jax.ShapeDtypeStruct((1, 1), jnp.float32),
        scratch_shapes=[
            pltpu.VMEM((2, _CHD, _NCH), jnp.float32),
            pltpu.SemaphoreType.DMA,
            pltpu.SemaphoreType.DMA,
        ],
    )(pt, gb, gl)
    return out[0, 0]


# speedup vs baseline: 8.7870x; 1.3184x over previous
import jax
import jax.numpy as jnp
from jax.experimental import pallas as pl
from jax.experimental.pallas import tpu as pltpu

_NUM_CLASSES = 20
_NUM_ANC = 5
_GRID = 13
_BATCH = 16
_NBOX = 8
_NB = _BATCH * _NBOX
_S = 1 + _NUM_CLASSES
_NCELL = _GRID * _GRID
_NROW = _NCELL * _BATCH
_NCH = (_S + 4) * _NUM_ANC
_EPS16 = 0.0009765625
_ANCW = (0.074, 0.147, 0.282, 0.471, 0.784)
_ANCH = (0.060, 0.151, 0.231, 0.425, 0.740)

_DNT = (((1,), (1,)), ((), ()))


def _loss_body(pt_ref, gb_ref, gl_ref, out_ref):
    f32 = jnp.float32
    i32 = jnp.int32
    lft = gb_ref[:, 0, :]
    top = gb_ref[:, 1, :]
    rgt = gb_ref[:, 2, :]
    btm = gb_ref[:, 3, :]
    glint = gl_ref[...]

    w_g = rgt - lft
    h_g = btm - top
    area_g = w_g * h_g
    best = jnp.full(w_g.shape, -1.0, f32)
    ids = jnp.zeros(w_g.shape, i32)
    for a in range(_NUM_ANC):
        inter = jnp.minimum(w_g, _ANCW[a]) * jnp.minimum(h_g, _ANCH[a])
        iou = inter / (area_g + _ANCW[a] * _ANCH[a] - inter)
        upd = iou > best
        ids = jnp.where(upd, a, ids)
        best = jnp.maximum(iou, best)

    xsx = (lft + rgt) * (0.5 * _GRID)
    xsy = (top + btm) * (0.5 * _GRID)
    flx = jnp.floor(xsx)
    fly = jnp.floor(xsy)
    txx = xsx - flx
    txyy = xsy - fly
    cell = fly.astype(i32) * _GRID + flx.astype(i32)
    aw = jnp.full(w_g.shape, _ANCW[-1], f32)
    ah = jnp.full(w_g.shape, _ANCH[-1], f32)
    for a in range(_NUM_ANC - 2, -1, -1):
        aw = jnp.where(ids == a, _ANCW[a], aw)
        ah = jnp.where(ids == a, _ANCH[a], ah)
    twx = jnp.log(w_g / aw)
    twy = jnp.log(h_g / ah)
    wgt = 2.0 - area_g

    ii = jax.lax.broadcasted_iota(i32, (_NBOX, _NBOX, _BATCH), 0)
    jj = jax.lax.broadcasted_iota(i32, (_NBOX, _NBOX, _BATCH), 1)
    clob = (cell[None, :, :] == cell[:, None, :]) & (jj > ii)
    last = jnp.where(jnp.any(clob, axis=1), 0.0, 1.0)

    bimg = jax.lax.broadcasted_iota(i32, (_NBOX, _BATCH), 1)
    hot = cell * _BATCH + bimg
    riota = jax.lax.broadcasted_iota(i32, (_NBOX, _BATCH, _NROW), 2)
    oh = jnp.reshape((riota == hot[..., None]).astype(f32), (_NB, _NROW))
    pt = pt_ref[...]
    gat2 = jax.lax.dot_general(oh, pt, (((1,), (0,)), ((), ())),
                               preferred_element_type=f32)
    gat = jnp.reshape(gat2, (_NBOX, _BATCH, _NCH))

    kio5 = jax.lax.broadcasted_iota(i32, (_NUM_ANC, _NCH), 1)
    ai5 = jax.lax.broadcasted_iota(i32, (_NUM_ANC, _NCH), 0)
    eye5 = (kio5 == ai5).astype(f32)
    confT = jax.lax.dot_general(eye5, pt, _DNT,
                                preferred_element_type=f32)
    pcA = jnp.clip(jax.nn.sigmoid(confT), 1e-7, 1.0 - 1e-7)
    fnegA = -0.5 * pcA * pcA * jnp.log(1.0 - pcA)
    ksum = jnp.sum(fnegA, axis=0, keepdims=True)
    rio = jax.lax.broadcasted_iota(i32, (_BATCH, _NROW), 1)
    bio = jax.lax.broadcasted_iota(i32, (_BATCH, _NROW), 0)
    selt = ((rio & (_BATCH - 1)) == bio).astype(f32)
    neg_all = jax.lax.dot_general(ksum, selt, _DNT,
                                  preferred_element_type=f32)

    kiota = jax.lax.broadcasted_iota(i32, (_NBOX, _BATCH, _NCH), 2)
    cmap = jnp.floor(kiota.astype(f32) * 0.2).astype(i32)
    amap = kiota - cmap * _NUM_ANC
    am = amap == ids[..., None]
    pcl = jnp.clip(jax.nn.sigmoid(gat), 1e-7, 1.0 - 1e-7)
    logp = jnp.log(pcl)
    log1mp = jnp.log(1.0 - pcl)

    is_conf = cmap == 0
    fneg = -0.5 * pcl * pcl * log1mp
    sub = jnp.sum(jnp.where(is_conf, fneg, 0.0), axis=2)
    fpos = -0.5 * (1.0 - pcl) * (1.0 - pcl) * logp
    cpos = jnp.sum(jnp.where(is_conf & am, fpos, 0.0), axis=2)

    tcls = (cmap == glint[..., None]).astype(f32)
    bce_cls = -(tcls * logp + (1.0 - tcls) * log1mp)
    mcls = (cmap >= 1) & (cmap <= _NUM_CLASSES) & am
    clsv = jnp.sum(jnp.where(mcls, bce_cls, 0.0), axis=2)

    tgxy = jnp.where(cmap == _S, txx[..., None], txyy[..., None])
    bce_xy = -(tgxy * logp + (1.0 - tgxy) * log1mp)
    mxy = ((cmap == _S) | (cmap == _S + 1)) & am
    xyv = jnp.sum(jnp.where(mxy, bce_xy, 0.0), axis=2)

    tgwh = jnp.where(cmap == _S + 2, twx[..., None], twy[..., None])
    dwh = gat - tgwh
    mwh = ((cmap == _S + 2) | (cmap == _S + 3)) & am
    whv = jnp.sum(jnp.where(mwh, dwh * dwh, 0.0), axis=2)

    npos_b = jnp.sum(last, axis=0, keepdims=True)
    sub_i = jnp.sum(last * sub, axis=0, keepdims=True)
    cpos_i = jnp.sum(last * cpos, axis=0, keepdims=True)
    npos_t = jnp.maximum(jnp.sum(npos_b), 1.0)
    nneg_b = jnp.maximum(float(_NCELL * _NUM_ANC) - 5.0 * npos_b, _EPS16)
    npos_bc = jnp.maximum(npos_b, _EPS16)
    l_conf_neg = jnp.sum((neg_all - sub_i) / nneg_b) * (3.0 / _BATCH)
    l_conf_pos = jnp.sum(cpos_i / npos_bc) / _BATCH
    l_cls = jnp.sum(last * clsv) / npos_t
    l_txty = jnp.sum(last * wgt * xyv) / npos_t
    l_twth = jnp.sum(last * wgt * whv) / npos_t

    total = l_conf_pos + l_conf_neg + l_cls + l_txty + l_twth
    out_ref[...] = jnp.broadcast_to(total, (1, 1))


def kernel(pyolos, gboxes_ltrb, glabels):
    pt = pyolos.transpose(2, 3, 0, 1).reshape(_NROW, _NCH)
    gb = gboxes_ltrb.astype(jnp.float32).transpose(1, 2, 0)
    gl = glabels.astype(jnp.int32).transpose(1, 0)
    out = pl.pallas_call(
        _loss_body,
        out_shape=jax.ShapeDtypeStruct((1, 1), jnp.float32),
    )(pt, gb, gl)
    return out[0, 0]
